# Initial kernel scaffold; baseline (speedup 1.0000x reference)
import jax
import jax.numpy as jnp
from jax import lax
from jax.experimental import pallas as pl
from jax.experimental.pallas import tpu as pltpu
from jax.experimental.pallas import tpu_sc as plsc

D = 128
L = 16
P = 128
NW = 32


def _body(w_in, w_out, b_in, b_out, i_idx, j_idx, out,
          i_v, j_v, wi_buf, wj_buf, bi_buf, bj_buf, accs, out_buf,
          sem0, sem1, sem2, sem3):
  n_per_w = i_idx.shape[0] // NW
  n_chunks = n_per_w // P
  wid = lax.axis_index("s") * 2 + lax.axis_index("c")
  base = wid * n_per_w
  iota = lax.iota(jnp.int32, L)

  def chunk_body(c, _):
    off = base + c * P
    pltpu.sync_copy(i_idx.at[pl.ds(off, P)], i_v)
    pltpu.sync_copy(j_idx.at[pl.ds(off, P)], j_v)
    cp0 = pltpu.async_copy(w_in.at[i_v], wi_buf, sem0)
    cp1 = pltpu.async_copy(w_out.at[j_v], wj_buf, sem1)
    cp2 = pltpu.async_copy(b_in.at[i_v], bi_buf, sem2)
    cp3 = pltpu.async_copy(b_out.at[j_v], bj_buf, sem3)
    cp0.wait()
    cp1.wait()
    cp2.wait()
    cp3.wait()

    def group_body(g, _):
      for u in range(L):
        p = g * L + u
        acc = wi_buf[p, pl.ds(0, L)] * wj_buf[p, pl.ds(0, L)]
        for k in range(1, D // L):
          acc += wi_buf[p, pl.ds(k * L, L)] * wj_buf[p, pl.ds(k * L, L)]
        accs[u, :] = acc
      out_v = bi_buf[pl.ds(g * L, L)] + bj_buf[pl.ds(g * L, L)]
      for l in range(L):
        out_v += plsc.load_gather(accs, [iota, jnp.full((L,), l, jnp.int32)])
      out_buf[pl.ds(g * L, L)] = out_v
      return 0

    lax.fori_loop(0, P // L, group_body, 0)
    pltpu.sync_copy(out_buf, out.at[pl.ds(off, P)])
    return 0

  lax.fori_loop(0, n_chunks, chunk_body, 0)


def kernel(words, W_in, W_out, bias_in, bias_out):
  if words.ndim == 1 and words.size == 2:
    words = words[None, :]
  B = words.shape[0]
  i_idx = words[:, 0]
  j_idx = words[:, 1]

  mesh = plsc.VectorSubcoreMesh(core_axis_name="c", subcore_axis_name="s")
  k = pl.kernel(
      _body,
      out_type=jax.ShapeDtypeStruct((B,), jnp.float32),
      mesh=mesh,
      scratch_types=[
          pltpu.VMEM((P,), jnp.int32),
          pltpu.VMEM((P,), jnp.int32),
          pltpu.VMEM((P, D), jnp.float32),
          pltpu.VMEM((P, D), jnp.float32),
          pltpu.VMEM((P,), jnp.float32),
          pltpu.VMEM((P,), jnp.float32),
          pltpu.VMEM((L, L), jnp.float32),
          pltpu.VMEM((P,), jnp.float32),
          pltpu.SemaphoreType.DMA,
          pltpu.SemaphoreType.DMA,
          pltpu.SemaphoreType.DMA,
          pltpu.SemaphoreType.DMA,
      ],
  )
  return k(W_in, W_out, bias_in, bias_out, i_idx, j_idx)


# speedup vs baseline: 1.5941x; 1.5941x over previous
import jax
import jax.numpy as jnp
from jax import lax
from jax.experimental import pallas as pl
from jax.experimental.pallas import tpu as pltpu
from jax.experimental.pallas import tpu_sc as plsc

D = 128
L = 16
P = 128
NW = 32
NBUF = 2


def _body(w_in, w_out, b_in, b_out, i_idx, j_idx, out,
          i_v, j_v, wi_buf, wj_buf, bi_buf, bj_buf, accs, out_buf, sems):
  n_per_w = i_idx.shape[0] // NW
  n_chunks = n_per_w // P
  wid = lax.axis_index("s") * 2 + lax.axis_index("c")
  base = wid * n_per_w
  iota = lax.iota(jnp.int32, L)

  def issue(c, s):
    off = base + c * P
    pltpu.sync_copy(i_idx.at[pl.ds(off, P)], i_v.at[s])
    pltpu.sync_copy(j_idx.at[pl.ds(off, P)], j_v.at[s])
    return (pltpu.async_copy(w_in.at[i_v.at[s]], wi_buf.at[s], sems.at[s, 0]),
            pltpu.async_copy(w_out.at[j_v.at[s]], wj_buf.at[s], sems.at[s, 1]),
            pltpu.async_copy(b_in.at[i_v.at[s]], bi_buf.at[s], sems.at[s, 2]),
            pltpu.async_copy(b_out.at[j_v.at[s]], bj_buf.at[s], sems.at[s, 3]))

  def compute(c, s):
    off = base + c * P

    def group_body(g, _):
      for u in range(L):
        p = g * L + u
        acc = wi_buf[s, p, pl.ds(0, L)] * wj_buf[s, p, pl.ds(0, L)]
        for k in range(1, D // L):
          acc += wi_buf[s, p, pl.ds(k * L, L)] * wj_buf[s, p, pl.ds(k * L, L)]
        accs[pl.ds(u * L, L)] = acc
      out_v = bi_buf[s, pl.ds(g * L, L)] + bj_buf[s, pl.ds(g * L, L)]
      for l in range(L):
        out_v += plsc.load_gather(accs, [iota * L + l])
      out_buf[pl.ds(g * L, L)] = out_v
      return 0

    lax.fori_loop(0, P // L, group_body, 0)
    pltpu.sync_copy(out_buf, out.at[pl.ds(off, P)])

  cps = issue(0, 0)
  for c in range(n_chunks):
    s = c % NBUF
    nxt = issue(c + 1, (c + 1) % NBUF) if c + 1 < n_chunks else None
    for cp in cps:
      cp.wait()
    compute(c, s)
    cps = nxt


def kernel(words, W_in, W_out, bias_in, bias_out):
  if words.ndim == 1 and words.size == 2:
    words = words[None, :]
  B = words.shape[0]
  i_idx = words[:, 0]
  j_idx = words[:, 1]

  mesh = plsc.VectorSubcoreMesh(
      core_axis_name="c", subcore_axis_name="s", num_cores=2, num_subcores=16)
  k = pl.kernel(
      _body,
      out_type=jax.ShapeDtypeStruct((B,), jnp.float32),
      mesh=mesh,
      compiler_params=pltpu.CompilerParams(needs_layout_passes=False),
      scratch_types=[
          pltpu.VMEM((NBUF, P), jnp.int32),
          pltpu.VMEM((NBUF, P), jnp.int32),
          pltpu.VMEM((NBUF, P, D), jnp.float32),
          pltpu.VMEM((NBUF, P, D), jnp.float32),
          pltpu.VMEM((NBUF, P), jnp.float32),
          pltpu.VMEM((NBUF, P), jnp.float32),
          pltpu.VMEM((L * L,), jnp.float32),
          pltpu.VMEM((P,), jnp.float32),
          pltpu.SemaphoreType.DMA((NBUF, 4)),
      ],
  )
  return k(W_in, W_out, bias_in, bias_out, i_idx, j_idx)
